# Initial kernel scaffold; baseline (speedup 1.0000x reference)
import jax
import jax.numpy as jnp
from jax.experimental import pallas as pl


def kernel(idx, features_packed, zbuf):
    raise NotImplementedError("write your pallas kernel here")


# speedup vs baseline: 1.5035x; 1.5035x over previous
import functools

import jax
import jax.numpy as jnp
from jax import lax
from jax.experimental import pallas as pl
from jax.experimental.pallas import tpu as pltpu
from jax.experimental.pallas import tpu_sc as plsc

NC, NS, L = 2, 16, 16
NW = NC * NS

CHUNK = 1024
GW = 128


def _renderer(idx_flat, table_aug, zbuf_flat, *, n, p, k, c):
    npix = n // NW
    nchunks = npix // CHUNK
    c1 = c + 1
    mesh = plsc.VectorSubcoreMesh(
        core_axis_name="c", subcore_axis_name="s",
        num_cores=NC, num_subcores=NS)

    @functools.partial(
        pl.kernel,
        out_type=(
            jax.ShapeDtypeStruct((n * c1,), jnp.float32),
            jax.ShapeDtypeStruct((n * k,), jnp.float32),
        ),
        mesh=mesh,
        compiler_params=pltpu.CompilerParams(
            needs_layout_passes=False, use_tc_tiling_on_sc=False),
        scratch_types=[
            pltpu.VMEM((CHUNK * k,), jnp.int32),
            pltpu.VMEM((CHUNK * k,), jnp.float32),
            pltpu.VMEM((CHUNK,), jnp.int32),
            pltpu.VMEM((L,), jnp.float32),
            pltpu.VMEM((CHUNK, c), jnp.float32),
            pltpu.VMEM((CHUNK * c1,), jnp.float32),
            pltpu.SemaphoreType.DMA,
        ],
    )
    def body(idx_hbm, tab_hbm, zb_hbm, feat_hbm, dep_hbm,
             idx_v, zb_v, sel_v, al_v, rows_v, packed_v, sem):
        wid = lax.axis_index("s") * NC + lax.axis_index("c")
        iota = lax.iota(jnp.int32, L)
        base_px = wid * npix

        def chunk_body(ci, carry):
            row0 = base_px + ci * CHUNK
            pltpu.sync_copy(idx_hbm.at[pl.ds(row0 * k, CHUNK * k)], idx_v)
            pltpu.sync_copy(zb_hbm.at[pl.ds(row0 * k, CHUNK * k)], zb_v)

            def grp(g, c2):
                iv = iota * k + g * (L * k)
                idx0 = plsc.load_gather(idx_v, [iv])
                m = idx0 >= 0
                alpha = jnp.where(m, 1.0, 0.0).astype(jnp.float32)
                sel = jnp.where(m, jnp.minimum(idx0, p - 1), p)
                sel_v[pl.ds(g * L, L)] = sel
                al_v[...] = alpha
                half = jnp.right_shift(iota, 3)
                for t in range(k):
                    mm = plsc.load_gather(al_v, [half + 2 * t])
                    off = g * (L * k) + t * L
                    zb_v[pl.ds(off, L)] = zb_v[pl.ds(off, L)] * mm
                plsc.store_scatter(
                    packed_v, [iota * c1 + (g * L * c1 + c)], alpha)
                return c2

            lax.fori_loop(0, CHUNK // L, grp, 0)
            pltpu.sync_copy(zb_v, dep_hbm.at[pl.ds(row0 * k, CHUNK * k)])

            copies = []
            for j in range(CHUNK // GW):
                copies.append(pltpu.async_copy(
                    tab_hbm.at[sel_v.at[pl.ds(j * GW, GW)]],
                    rows_v.at[pl.ds(j * GW, GW), :],
                    sem))
            for cp in copies:
                cp.wait()

            def pack(g, c2):
                for r in range(L):
                    row = g * L + r
                    for h in range(c // L):
                        packed_v[pl.ds(row * c1 + h * L, L)] = (
                            rows_v[row, pl.ds(h * L, L)])
                return c2

            lax.fori_loop(0, CHUNK // L, pack, 0)
            pltpu.sync_copy(packed_v,
                            feat_hbm.at[pl.ds(row0 * c1, CHUNK * c1)])
            return carry

        lax.fori_loop(0, nchunks, chunk_body, 0)

    return body(idx_flat, table_aug, zbuf_flat)


def kernel(idx, features_packed, zbuf):
    B, H, W, K = idx.shape
    P, C = features_packed.shape
    n = B * H * W
    table_aug = jnp.concatenate(
        [features_packed, jnp.zeros((1, C), jnp.float32)], axis=0)
    idx_flat = idx.reshape(n * K)
    zbuf_flat = zbuf.reshape(n * K)
    feat, dep = _renderer(idx_flat, table_aug, zbuf_flat,
                          n=n, p=P, k=K, c=C)
    return feat.reshape(B, H, W, C + 1), dep.reshape(B, H, W, K)


# speedup vs baseline: 4.1601x; 2.7670x over previous
import functools

import jax
import jax.numpy as jnp
from jax import lax
from jax.experimental import pallas as pl
from jax.experimental.pallas import tpu as pltpu
from jax.experimental.pallas import tpu_sc as plsc

NC, NS, L = 2, 16, 16

B, H, W, K, P, C = 4, 512, 512, 8, 100000, 32
HT = H // 8
WT = W // 128
TPI = HT * WT
PLANE = TPI * 1024

SEL_STEP = 2048
ZROWS = 4


def _renderer(idx_k, tab_t, zb_k):
    mesh = plsc.VectorSubcoreMesh(
        core_axis_name="c", subcore_axis_name="s",
        num_cores=NC, num_subcores=NS)

    @functools.partial(
        pl.kernel,
        out_type=(
            jax.ShapeDtypeStruct((B * (C + 1), TPI, 1024), jnp.float32),
            jax.ShapeDtypeStruct((B * H, WT, K, 128), jnp.float32),
        ),
        mesh=mesh,
        compiler_params=pltpu.CompilerParams(
            needs_layout_passes=False, use_tc_tiling_on_sc=False),
        scratch_types=[
            pltpu.VMEM((P + L,), jnp.float32),
            pltpu.VMEM((8, 128), jnp.int32),
            pltpu.VMEM((1024,), jnp.int32),
            pltpu.VMEM((1024,), jnp.float32),
            pltpu.VMEM((ZROWS, 8, 128), jnp.float32),
            pltpu.VMEM((SEL_STEP,), jnp.int32),
            pltpu.VMEM((SEL_STEP // 1024, 1024), jnp.float32),
            pltpu.VMEM_SHARED((PLANE,), jnp.int32),
            pltpu.SemaphoreType.DMA,
            pltpu.SemaphoreType.DMA,
        ],
    )
    def body(idx_hbm, tab_hbm, zb_hbm, feat_hbm, dep_hbm,
             plane_v, idx0_v, sel_v, al_v, zb_v, selb_v, ob_v, sel_sh,
             psem, sem):
        core = lax.axis_index("c")
        tile = lax.axis_index("s")
        chunks_per_tile = TPI // NS

        first_cp = pltpu.async_copy(
            tab_hbm.at[tile, :], plane_v.at[pl.ds(0, P)], psem)

        def phase1(b, carry):
            def chunk(ci, c1):
                bh0 = b * H + (ci // WT) * 8
                wt = ci % WT
                pltpu.sync_copy(idx_hbm.at[pl.ds(bh0, 8), wt, 0], idx0_v)

                def grp(g, c2):
                    hs = jnp.right_shift(g, 3)
                    wg = jnp.bitwise_and(g, 7) * L
                    idx0 = idx0_v[hs, pl.ds(wg, L)]
                    m = idx0 >= 0
                    sel_v[pl.ds(g * L, L)] = jnp.where(
                        m, jnp.minimum(idx0, P - 1), P)
                    al_v[pl.ds(g * L, L)] = m.astype(jnp.float32)
                    return c2

                lax.fori_loop(0, 64, grp, 0)
                pltpu.sync_copy(sel_v, sel_sh.at[pl.ds(ci * 1024, 1024)])
                pltpu.sync_copy(al_v, feat_hbm.at[b * (C + 1) + C, ci, :])

                for half in range(8 // ZROWS):
                    r0 = half * ZROWS
                    pltpu.sync_copy(
                        zb_hbm.at[pl.ds(bh0 + r0, ZROWS), wt], zb_v)

                    def dgrp(g, c2):
                        hs = jnp.right_shift(g, 3)
                        wg = jnp.bitwise_and(g, 7) * L
                        mm = al_v[pl.ds((r0 + hs) * 128 + wg, L)]
                        for kk in range(K):
                            zb_v[hs, kk, pl.ds(wg, L)] = (
                                zb_v[hs, kk, pl.ds(wg, L)] * mm)
                        return c2

                    lax.fori_loop(0, 8 * ZROWS, dgrp, 0)
                    pltpu.sync_copy(
                        zb_v, dep_hbm.at[pl.ds(bh0 + r0, ZROWS), wt])
                return c1

            lax.fori_loop(tile * chunks_per_tile,
                          (tile + 1) * chunks_per_tile, chunk, 0)
            return carry

        def phase2(b, chan, cp, carry):
            cp.wait()
            plane_v[pl.ds(P, L)] = jnp.zeros((L,), jnp.float32)
            row = b * (C + 1) + chan
            nrows = SEL_STEP // 1024

            def step(si, c1):
                pltpu.sync_copy(
                    sel_sh.at[pl.ds(si * SEL_STEP, SEL_STEP)], selb_v)

                def grp(g, c2):
                    s = selb_v[pl.ds(g * L, L)]
                    v = plsc.load_gather(plane_v, [s])
                    ob_v[jnp.right_shift(g, 6),
                         pl.ds(jnp.bitwise_and(g, 63) * L, L)] = v
                    return c2

                lax.fori_loop(0, SEL_STEP // L, grp, 0)
                pltpu.sync_copy(
                    ob_v, feat_hbm.at[row, pl.ds(si * nrows, nrows), :])
                return c1

            lax.fori_loop(0, PLANE // SEL_STEP, step, 0)
            return carry

        def img(img_i, cp):
            b = NC * core + img_i
            phase1(b, 0)
            plsc.subcore_barrier()
            phase2(b, tile, cp, 0)
            cp2 = pltpu.async_copy(
                tab_hbm.at[NS + tile, :], plane_v.at[pl.ds(0, P)], psem)
            phase2(b, NS + tile, cp2, 0)
            plsc.subcore_barrier()
            return pltpu.async_copy(
                tab_hbm.at[tile, :], plane_v.at[pl.ds(0, P)], psem)

        cp = first_cp
        for img_i in range(NC):
            cp = img(img_i, cp)
        cp.wait()

    return body(idx_k, tab_t, zb_k)


def kernel(idx, features_packed, zbuf):
    idx_k = idx.reshape(B * H, WT, 128, K).transpose(0, 1, 3, 2)
    zb_k = zbuf.reshape(B * H, WT, 128, K).transpose(0, 1, 3, 2)
    tab_t = features_packed.T
    feat_p, dep_k = _renderer(idx_k, tab_t, zb_k)
    feat = (feat_p.reshape(B, C + 1, HT, WT, 8, 128)
            .transpose(0, 2, 4, 3, 5, 1)
            .reshape(B, H, W, C + 1))
    dep = (dep_k.reshape(B * H, WT, K, 128)
           .transpose(0, 1, 3, 2)
           .reshape(B, H, W, K))
    return feat, dep


# speedup vs baseline: 7.3127x; 1.7578x over previous
import functools

import jax
import jax.numpy as jnp
from jax import lax
from jax.experimental import pallas as pl
from jax.experimental.pallas import tpu as pltpu
from jax.experimental.pallas import tpu_sc as plsc

NC, NS, L = 2, 16, 16

B, H, W, K, P, C = 4, 512, 512, 8, 100000, 32
HT = H // 8
WT = W // 128
TPI = HT * WT
PLANE = TPI * 1024

STEP = 1024
NSTEPS = PLANE // STEP
ZROWS = 2


def _renderer(idx_k, tab_t, zb_k):
    mesh = plsc.VectorSubcoreMesh(
        core_axis_name="c", subcore_axis_name="s",
        num_cores=NC, num_subcores=NS)

    @functools.partial(
        pl.kernel,
        out_type=(
            jax.ShapeDtypeStruct((B * (C + 1), TPI, 1024), jnp.float32),
            jax.ShapeDtypeStruct((B * H, WT, K, 128), jnp.float32),
        ),
        mesh=mesh,
        compiler_params=pltpu.CompilerParams(
            needs_layout_passes=False, use_tc_tiling_on_sc=False),
        scratch_types=[
            pltpu.VMEM((P + L,), jnp.float32),
            pltpu.VMEM((8, 128), jnp.int32),
            pltpu.VMEM((1024,), jnp.int32),
            pltpu.VMEM((2, ZROWS, 8, 128), jnp.float32),
            pltpu.VMEM((2, STEP), jnp.int32),
            pltpu.VMEM((2, STEP), jnp.float32),
            pltpu.VMEM_SHARED((PLANE,), jnp.int32),
            pltpu.SemaphoreType.DMA,
            pltpu.SemaphoreType.DMA,
            pltpu.SemaphoreType.DMA,
            pltpu.SemaphoreType.DMA,
            pltpu.SemaphoreType.DMA,
            pltpu.SemaphoreType.DMA,
            pltpu.SemaphoreType.DMA,
        ],
    )
    def body(idx_hbm, tab_hbm, zb_hbm, feat_hbm, dep_hbm,
             plane_v, idx0_v, sel_v, zb_v, selb_v, ob_v, sel_sh,
             psem, zs0, zs1, ss0, ss1, os0, os1):
        core = lax.axis_index("c")
        tile = lax.axis_index("s")
        zsems = (zs0, zs1)
        ssems = (ss0, ss1)
        osems = (os0, os1)
        cpt = TPI // NS

        first_cp = pltpu.async_copy(
            tab_hbm.at[tile, :], plane_v.at[pl.ds(0, P)], psem)

        def phase1(b):
            c0 = tile * cpt

            def chunk(ci, carry):
                bh0 = b * H + (ci // WT) * 8
                wt = ci % WT
                pltpu.sync_copy(idx_hbm.at[pl.ds(bh0, 8), wt, 0], idx0_v)

                def grp(g8, c2):
                    for u in range(8):
                        g = g8 * 8 + u
                        hs = jnp.right_shift(g, 3)
                        wg = jnp.bitwise_and(g, 7) * L
                        idx0 = idx0_v[hs, pl.ds(wg, L)]
                        sel_v[pl.ds(g * L, L)] = jnp.where(
                            idx0 >= 0, jnp.minimum(idx0, P - 1), P)
                    return c2

                lax.fori_loop(0, 8, grp, 0)
                pltpu.sync_copy(sel_v, sel_sh.at[pl.ds(ci * 1024, 1024)])

                for piece in range(8 // ZROWS):
                    j = piece % 2
                    r0 = piece * ZROWS

                    def wait_zb():
                        pltpu.make_async_copy(
                            zb_v.at[j],
                            dep_hbm.at[pl.ds(bh0 + r0, ZROWS), wt],
                            zsems[j]).wait()

                    if piece >= 2:
                        wait_zb()
                    else:
                        pl.when(ci > c0)(wait_zb)

                    pltpu.sync_copy(
                        zb_hbm.at[pl.ds(bh0 + r0, ZROWS), wt], zb_v.at[j])

                    def dgrp(gd, c2):
                        for u in range(4):
                            g = gd * 4 + u
                            hs = jnp.right_shift(g, 3)
                            wg = jnp.bitwise_and(g, 7) * L
                            mm = (idx0_v[r0 + hs, pl.ds(wg, L)] >= 0
                                  ).astype(jnp.float32)
                            for kk in range(K):
                                zb_v[j, hs, kk, pl.ds(wg, L)] = (
                                    zb_v[j, hs, kk, pl.ds(wg, L)] * mm)
                        return c2

                    lax.fori_loop(0, ZROWS * 8 // 4, dgrp, 0)
                    pltpu.async_copy(
                        zb_v.at[j],
                        dep_hbm.at[pl.ds(bh0 + r0, ZROWS), wt], zsems[j])
                return carry

            lax.fori_loop(c0, c0 + cpt, chunk, 0)
            for j in range(2):
                pltpu.make_async_copy(
                    zb_v.at[j], dep_hbm.at[pl.ds(0, ZROWS), 0],
                    zsems[j]).wait()

        def start_sel(si, j):
            pltpu.async_copy(
                sel_sh.at[pl.ds(si * STEP, STEP)], selb_v.at[j], ssems[j])

        def gather_steps(row, s0, nsteps, compute):
            start_sel(s0, 0)
            start_sel(s0 + 1, 1)

            def pair(sp, carry):
                for j in range(2):
                    si = s0 + sp * 2 + j
                    pltpu.make_async_copy(
                        sel_sh.at[pl.ds(si * STEP, STEP)], selb_v.at[j],
                        ssems[j]).wait()

                    @pl.when(sp > 0)
                    def _():
                        pltpu.make_async_copy(
                            ob_v.at[j], feat_hbm.at[row, si - 2, :],
                            osems[j]).wait()

                    compute(j)

                    @pl.when(si + 2 < s0 + nsteps)
                    def _():
                        start_sel(si + 2, j)

                    pltpu.async_copy(
                        ob_v.at[j], feat_hbm.at[row, si, :], osems[j])
                return carry

            lax.fori_loop(0, nsteps // 2, pair, 0)
            for j in range(2):
                pltpu.make_async_copy(
                    ob_v.at[j], feat_hbm.at[row, 0, :], osems[j]).wait()

        def chan_compute(j):
            def grp(g8, c2):
                for u in range(8):
                    g = g8 * 8 + u
                    s = selb_v[j, pl.ds(g * L, L)]
                    ob_v[j, pl.ds(g * L, L)] = plsc.load_gather(
                        plane_v, [s])
                return c2
            lax.fori_loop(0, STEP // L // 8, grp, 0)

        def alpha_compute(j):
            def grp(g8, c2):
                for u in range(8):
                    g = g8 * 8 + u
                    s = selb_v[j, pl.ds(g * L, L)]
                    ob_v[j, pl.ds(g * L, L)] = (s < P).astype(jnp.float32)
                return c2
            lax.fori_loop(0, STEP // L // 8, grp, 0)

        def phase2(b, chan, cp):
            cp.wait()
            plane_v[pl.ds(P, L)] = jnp.zeros((L,), jnp.float32)
            gather_steps(b * (C + 1) + chan, 0, NSTEPS, chan_compute)

        def img(img_i, cp):
            b = NC * core + img_i
            phase1(b)
            plsc.subcore_barrier()
            phase2(b, tile, cp)
            cp2 = pltpu.async_copy(
                tab_hbm.at[NS + tile, :], plane_v.at[pl.ds(0, P)], psem)
            phase2(b, NS + tile, cp2)
            gather_steps(b * (C + 1) + C, tile * (NSTEPS // NS),
                         NSTEPS // NS, alpha_compute)
            plsc.subcore_barrier()
            return pltpu.async_copy(
                tab_hbm.at[tile, :], plane_v.at[pl.ds(0, P)], psem)

        cp = first_cp
        for img_i in range(NC):
            cp = img(img_i, cp)
        cp.wait()

    return body(idx_k, tab_t, zb_k)


def kernel(idx, features_packed, zbuf):
    idx_k = idx.reshape(B * H, WT, 128, K).transpose(0, 1, 3, 2)
    zb_k = zbuf.reshape(B * H, WT, 128, K).transpose(0, 1, 3, 2)
    tab_t = features_packed.T
    feat_p, dep_k = _renderer(idx_k, tab_t, zb_k)
    feat = (feat_p.reshape(B, C + 1, HT, WT, 8, 128)
            .transpose(0, 2, 4, 3, 5, 1)
            .reshape(B, H, W, C + 1))
    dep = (dep_k.reshape(B * H, WT, K, 128)
           .transpose(0, 1, 3, 2)
           .reshape(B, H, W, K))
    return feat, dep


# speedup vs baseline: 8.6339x; 1.1807x over previous
import functools

import jax
import jax.numpy as jnp
from jax import lax
from jax.experimental import pallas as pl
from jax.experimental.pallas import tpu as pltpu
from jax.experimental.pallas import tpu_sc as plsc

NC, NS, L = 2, 16, 16

B, H, W, K, P, C = 4, 512, 512, 8, 100000, 32
HT = H // 8
WT = W // 128
TPI = HT * WT
PLANE = TPI * 1024

STEP = 2048
NSTEPS = PLANE // STEP


def _renderer(idx_k, tab_t, zb_k):
    mesh = plsc.VectorSubcoreMesh(
        core_axis_name="c", subcore_axis_name="s",
        num_cores=NC, num_subcores=NS)

    @functools.partial(
        pl.kernel,
        out_type=(
            jax.ShapeDtypeStruct((B * (C + 1), PLANE), jnp.float32),
            jax.ShapeDtypeStruct((B * H, WT, K, 128), jnp.float32),
        ),
        mesh=mesh,
        compiler_params=pltpu.CompilerParams(
            needs_layout_passes=False, use_tc_tiling_on_sc=False),
        scratch_types=[
            pltpu.VMEM((P + L,), jnp.float32),
            pltpu.VMEM((8, 128), jnp.int32),
            pltpu.VMEM((1024,), jnp.int32),
            pltpu.VMEM((2, 8, 128), jnp.float32),
            pltpu.VMEM((2, 8, 128), jnp.float32),
            pltpu.VMEM((2, STEP), jnp.int32),
            pltpu.VMEM((2, STEP), jnp.float32),
            pltpu.VMEM_SHARED((PLANE,), jnp.int32),
            pltpu.SemaphoreType.DMA,
            pltpu.SemaphoreType.DMA,
            pltpu.SemaphoreType.DMA,
            pltpu.SemaphoreType.DMA,
            pltpu.SemaphoreType.DMA,
            pltpu.SemaphoreType.DMA,
            pltpu.SemaphoreType.DMA,
            pltpu.SemaphoreType.DMA,
            pltpu.SemaphoreType.DMA,
            pltpu.SemaphoreType.DMA,
        ],
    )
    def body(idx_hbm, tab_hbm, zb_hbm, feat_hbm, dep_hbm,
             plane_v, idx0_v, sel_v, zr_v, zw_v, selb_v, ob_v, sel_sh,
             psem, isem, rs0, rs1, ws0, ws1, ss0, ss1, os0, os1):
        core = lax.axis_index("c")
        tile = lax.axis_index("s")
        rsems = (rs0, rs1)
        wsems = (ws0, ws1)
        ssems = (ss0, ss1)
        osems = (os0, os1)
        cpt = TPI // NS

        first_cp = pltpu.async_copy(
            tab_hbm.at[tile, :], plane_v.at[pl.ds(0, P)], psem)

        def bh_wt(b, ci):
            return b * H + (ci // WT) * 8, ci % WT

        def idx_read(b, ci):
            bh0, wt = bh_wt(b, ci)
            return pltpu.make_async_copy(
                idx_hbm.at[pl.ds(bh0, 8), wt, 0], idx0_v, isem)

        def z_read(b, ci, r, j):
            bh0, wt = bh_wt(b, ci)
            return pltpu.make_async_copy(
                zb_hbm.at[bh0 + r, wt], zr_v.at[j], rsems[j])

        def z_write(b, ci, r, j):
            bh0, wt = bh_wt(b, ci)
            return pltpu.make_async_copy(
                zw_v.at[j], dep_hbm.at[bh0 + r, wt], wsems[j])

        def phase1(b):
            c0 = tile * cpt
            arow = b * (C + 1) + C
            idx_read(b, c0).start()
            for j in range(2):
                z_read(b, c0, j, j).start()

            def chunk(ci, carry):
                idx_read(b, ci).wait()

                pl.when(ci > c0)(
                    lambda: pltpu.make_async_copy(
                        ob_v.at[0, pl.ds(0, 1024)],
                        feat_hbm.at[arow, pl.ds(0, 1024)],
                        osems[0]).wait())

                def grp(g8, c2):
                    for u in range(8):
                        g = g8 * 8 + u
                        hs = jnp.right_shift(g, 3)
                        wg = jnp.bitwise_and(g, 7) * L
                        idx0 = idx0_v[hs, pl.ds(wg, L)]
                        m = idx0 >= 0
                        sel_v[pl.ds(g * L, L)] = jnp.where(
                            m, jnp.minimum(idx0, P - 1), P)
                        ob_v[0, pl.ds(g * L, L)] = m.astype(jnp.float32)
                    return c2

                lax.fori_loop(0, 8, grp, 0)
                pltpu.sync_copy(sel_v, sel_sh.at[pl.ds(ci * 1024, 1024)])
                pltpu.async_copy(
                    ob_v.at[0, pl.ds(0, 1024)],
                    feat_hbm.at[arow, pl.ds(ci * 1024, 1024)], osems[0])
                pl.when(ci + 1 < c0 + cpt)(
                    lambda: idx_read(b, ci + 1).start())

                for r in range(8):
                    j = r % 2
                    z_read(b, ci, r, j).wait()
                    if r >= 2:
                        z_write(b, ci, r - 2, j).wait()
                    else:
                        pl.when(ci > c0)(
                            lambda: z_write(b, ci - 1, 6 + r, j).wait())

                    for g in range(8):
                        wg = g * L
                        mm = (sel_v[pl.ds(r * 128 + wg, L)] < P
                              ).astype(jnp.float32)
                        for kk in range(K):
                            zw_v[j, kk, pl.ds(wg, L)] = (
                                zr_v[j, kk, pl.ds(wg, L)] * mm)
                    z_write(b, ci, r, j).start()
                    if r < 6:
                        z_read(b, ci, r + 2, j).start()
                    else:
                        pl.when(ci + 1 < c0 + cpt)(
                            lambda: z_read(b, ci + 1, r - 6, j).start())
                return carry

            lax.fori_loop(c0, c0 + cpt, chunk, 0)
            for j in range(2):
                z_write(b, c0, 6 + j, j).wait()
            pltpu.make_async_copy(
                ob_v.at[0, pl.ds(0, 1024)],
                feat_hbm.at[arow, pl.ds(0, 1024)], osems[0]).wait()

        def start_sel(si, j):
            pltpu.async_copy(
                sel_sh.at[pl.ds(si * STEP, STEP)], selb_v.at[j], ssems[j])

        def gather_steps(row):
            start_sel(0, 0)
            start_sel(1, 1)

            def pair(sp, carry):
                for j in range(2):
                    si = sp * 2 + j
                    pltpu.make_async_copy(
                        sel_sh.at[pl.ds(si * STEP, STEP)], selb_v.at[j],
                        ssems[j]).wait()

                    @pl.when(sp > 0)
                    def _():
                        pltpu.make_async_copy(
                            ob_v.at[j],
                            feat_hbm.at[row, pl.ds(0, STEP)],
                            osems[j]).wait()

                    def grp(g8, c2):
                        for u in range(8):
                            g = g8 * 8 + u
                            s = selb_v[j, pl.ds(g * L, L)]
                            ob_v[j, pl.ds(g * L, L)] = (
                                plsc.load_gather(plane_v, [s]))
                        return c2

                    lax.fori_loop(0, STEP // L // 8, grp, 0)

                    @pl.when(si + 2 < NSTEPS)
                    def _():
                        start_sel(si + 2, j)

                    pltpu.async_copy(
                        ob_v.at[j],
                        feat_hbm.at[row, pl.ds(si * STEP, STEP)],
                        osems[j])
                return carry

            lax.fori_loop(0, NSTEPS // 2, pair, 0)
            for j in range(2):
                pltpu.make_async_copy(
                    ob_v.at[j], feat_hbm.at[row, pl.ds(0, STEP)],
                    osems[j]).wait()

        def phase2(b, chan, cp):
            if cp is not None:
                cp.wait()
                plane_v[pl.ds(P, L)] = jnp.zeros((L,), jnp.float32)
            gather_steps(b * (C + 1) + chan)

        b0 = NC * core
        b1 = b0 + 1

        phase1(b0)
        plsc.subcore_barrier()
        phase2(b0, tile, first_cp)
        cp2 = pltpu.async_copy(
            tab_hbm.at[NS + tile, :], plane_v.at[pl.ds(0, P)], psem)
        phase2(b0, NS + tile, cp2)
        plsc.subcore_barrier()
        phase1(b1)
        plsc.subcore_barrier()
        phase2(b1, NS + tile, None)
        cp3 = pltpu.async_copy(
            tab_hbm.at[tile, :], plane_v.at[pl.ds(0, P)], psem)
        phase2(b1, tile, cp3)

    return body(idx_k, tab_t, zb_k)


def kernel(idx, features_packed, zbuf):
    idx_k = idx.reshape(B * H, WT, 128, K).transpose(0, 1, 3, 2)
    zb_k = zbuf.reshape(B * H, WT, 128, K).transpose(0, 1, 3, 2)
    tab_t = features_packed.T
    feat_p, dep_k = _renderer(idx_k, tab_t, zb_k)
    feat = (feat_p.reshape(B, C + 1, HT, WT, 8, 128)
            .transpose(0, 2, 4, 3, 5, 1)
            .reshape(B, H, W, C + 1))
    dep = (dep_k.reshape(B * H, WT, K, 128)
           .transpose(0, 1, 3, 2)
           .reshape(B, H, W, K))
    return feat, dep


# speedup vs baseline: 9.9106x; 1.1479x over previous
import functools

import jax
import jax.numpy as jnp
from jax import lax
from jax.experimental import pallas as pl
from jax.experimental.pallas import tpu as pltpu
from jax.experimental.pallas import tpu_sc as plsc

NC, NS, L = 2, 16, 16

B, H, W, K, P, C = 4, 512, 512, 8, 100000, 32
HT = H // 8
WT = W // 128
TPI = HT * WT
PLANE = TPI * 1024

STEP = 2048
NSTEPS = PLANE // STEP

ZROWS_TC = 32


def _depth_tc(idx0_k, zb_k):
    def body(i_ref, z_ref, o_ref):
        m = (i_ref[...] >= 0).astype(jnp.float32)
        o_ref[...] = z_ref[...] * m[:, :, None, :]

    return pl.pallas_call(
        body,
        grid=(B * H // ZROWS_TC,),
        in_specs=[
            pl.BlockSpec((ZROWS_TC, WT, 128), lambda i: (i, 0, 0)),
            pl.BlockSpec((ZROWS_TC, WT, K, 128), lambda i: (i, 0, 0, 0)),
        ],
        out_specs=pl.BlockSpec((ZROWS_TC, WT, K, 128), lambda i: (i, 0, 0, 0)),
        out_shape=jax.ShapeDtypeStruct((B * H, WT, K, 128), jnp.float32),
    )(idx0_k, zb_k)


def _renderer(idx0_k, tab_t):
    mesh = plsc.VectorSubcoreMesh(
        core_axis_name="c", subcore_axis_name="s",
        num_cores=NC, num_subcores=NS)

    @functools.partial(
        pl.kernel,
        out_type=jax.ShapeDtypeStruct((B * (C + 1), PLANE), jnp.float32),
        mesh=mesh,
        compiler_params=pltpu.CompilerParams(
            needs_layout_passes=False, use_tc_tiling_on_sc=False),
        scratch_types=[
            pltpu.VMEM((P + L,), jnp.float32),
            pltpu.VMEM((8, 128), jnp.int32),
            pltpu.VMEM((1024,), jnp.int32),
            pltpu.VMEM((2, STEP), jnp.int32),
            pltpu.VMEM((2, STEP), jnp.float32),
            pltpu.VMEM_SHARED((PLANE,), jnp.int32),
            pltpu.SemaphoreType.DMA,
            pltpu.SemaphoreType.DMA,
            pltpu.SemaphoreType.DMA,
            pltpu.SemaphoreType.DMA,
            pltpu.SemaphoreType.DMA,
            pltpu.SemaphoreType.DMA,
        ],
    )
    def body(idx_hbm, tab_hbm, feat_hbm,
             plane_v, idx0_v, sel_v, selb_v, ob_v, sel_sh,
             psem, isem, ss0, ss1, os0, os1):
        core = lax.axis_index("c")
        tile = lax.axis_index("s")
        ssems = (ss0, ss1)
        osems = (os0, os1)
        cpt = TPI // NS

        first_cp = pltpu.async_copy(
            tab_hbm.at[tile, :], plane_v.at[pl.ds(0, P)], psem)

        def idx_read(b, ci):
            bh0 = b * H + (ci // WT) * 8
            return pltpu.make_async_copy(
                idx_hbm.at[pl.ds(bh0, 8), ci % WT], idx0_v, isem)

        def phase1(b):
            c0 = tile * cpt
            arow = b * (C + 1) + C
            idx_read(b, c0).start()

            def chunk(ci, carry):
                idx_read(b, ci).wait()
                pl.when(ci > c0)(
                    lambda: pltpu.make_async_copy(
                        ob_v.at[0, pl.ds(0, 1024)],
                        feat_hbm.at[arow, pl.ds(0, 1024)],
                        osems[0]).wait())

                def grp(g8, c2):
                    for u in range(8):
                        g = g8 * 8 + u
                        hs = jnp.right_shift(g, 3)
                        wg = jnp.bitwise_and(g, 7) * L
                        idx0 = idx0_v[hs, pl.ds(wg, L)]
                        m = idx0 >= 0
                        sel_v[pl.ds(g * L, L)] = jnp.where(
                            m, jnp.minimum(idx0, P - 1), P)
                        ob_v[0, pl.ds(g * L, L)] = m.astype(jnp.float32)
                    return c2

                lax.fori_loop(0, 8, grp, 0)
                pltpu.sync_copy(sel_v, sel_sh.at[pl.ds(ci * 1024, 1024)])
                pltpu.async_copy(
                    ob_v.at[0, pl.ds(0, 1024)],
                    feat_hbm.at[arow, pl.ds(ci * 1024, 1024)], osems[0])
                pl.when(ci + 1 < c0 + cpt)(
                    lambda: idx_read(b, ci + 1).start())
                return carry

            lax.fori_loop(c0, c0 + cpt, chunk, 0)
            pltpu.make_async_copy(
                ob_v.at[0, pl.ds(0, 1024)],
                feat_hbm.at[arow, pl.ds(0, 1024)], osems[0]).wait()

        def start_sel(si, j):
            pltpu.async_copy(
                sel_sh.at[pl.ds(si * STEP, STEP)], selb_v.at[j], ssems[j])

        def gather_steps(row):
            start_sel(0, 0)
            start_sel(1, 1)

            def pair(sp, carry):
                for j in range(2):
                    si = sp * 2 + j
                    pltpu.make_async_copy(
                        sel_sh.at[pl.ds(si * STEP, STEP)], selb_v.at[j],
                        ssems[j]).wait()

                    @pl.when(sp > 0)
                    def _():
                        pltpu.make_async_copy(
                            ob_v.at[j],
                            feat_hbm.at[row, pl.ds(0, STEP)],
                            osems[j]).wait()

                    def grp(g8, c2):
                        for u in range(8):
                            g = g8 * 8 + u
                            s = selb_v[j, pl.ds(g * L, L)]
                            ob_v[j, pl.ds(g * L, L)] = (
                                plsc.load_gather(plane_v, [s]))
                        return c2

                    lax.fori_loop(0, STEP // L // 8, grp, 0)

                    @pl.when(si + 2 < NSTEPS)
                    def _():
                        start_sel(si + 2, j)

                    pltpu.async_copy(
                        ob_v.at[j],
                        feat_hbm.at[row, pl.ds(si * STEP, STEP)],
                        osems[j])
                return carry

            lax.fori_loop(0, NSTEPS // 2, pair, 0)
            for j in range(2):
                pltpu.make_async_copy(
                    ob_v.at[j], feat_hbm.at[row, pl.ds(0, STEP)],
                    osems[j]).wait()

        def phase2(b, chan, cp):
            if cp is not None:
                cp.wait()
                plane_v[pl.ds(P, L)] = jnp.zeros((L,), jnp.float32)
            gather_steps(b * (C + 1) + chan)

        b0 = NC * core
        b1 = b0 + 1

        phase1(b0)
        plsc.subcore_barrier()
        phase2(b0, tile, first_cp)
        cp2 = pltpu.async_copy(
            tab_hbm.at[NS + tile, :], plane_v.at[pl.ds(0, P)], psem)
        phase2(b0, NS + tile, cp2)
        plsc.subcore_barrier()
        phase1(b1)
        plsc.subcore_barrier()
        phase2(b1, NS + tile, None)
        cp3 = pltpu.async_copy(
            tab_hbm.at[tile, :], plane_v.at[pl.ds(0, P)], psem)
        phase2(b1, tile, cp3)

    return body(idx0_k, tab_t)


def kernel(idx, features_packed, zbuf):
    idx0_k = idx[..., 0].reshape(B * H, WT, 128)
    zb_k = zbuf.reshape(B * H, WT, 128, K).transpose(0, 1, 3, 2)
    tab_t = features_packed.T
    dep_k = _depth_tc(idx0_k, zb_k)
    feat_p = _renderer(idx0_k, tab_t)
    feat = (feat_p.reshape(B, C + 1, HT, WT, 8, 128)
            .transpose(0, 2, 4, 3, 5, 1)
            .reshape(B, H, W, C + 1))
    dep = (dep_k.reshape(B * H, WT, K, 128)
           .transpose(0, 1, 3, 2)
           .reshape(B, H, W, K))
    return feat, dep


# speedup vs baseline: 10.2327x; 1.0325x over previous
import functools

import jax
import jax.numpy as jnp
from jax import lax
from jax.experimental import pallas as pl
from jax.experimental.pallas import tpu as pltpu
from jax.experimental.pallas import tpu_sc as plsc

NC, NS, L = 2, 16, 16

B, H, W, K, P, C = 4, 512, 512, 8, 100000, 32
HT = H // 8
WT = W // 128
TPI = HT * WT
PLANE = TPI * 1024

STEP = 2048
NSTEPS = PLANE // STEP

ZROWS_TC = 32


def _depth_tc(idx_k, zb_k):
    def body(i_ref, z_ref, o_ref):
        m = (i_ref[:, :, 0:1, :] >= 0).astype(jnp.float32)
        o_ref[...] = z_ref[...] * m

    return pl.pallas_call(
        body,
        grid=(B * H // ZROWS_TC,),
        in_specs=[
            pl.BlockSpec((ZROWS_TC, WT, K, 128), lambda i: (i, 0, 0, 0)),
            pl.BlockSpec((ZROWS_TC, WT, K, 128), lambda i: (i, 0, 0, 0)),
        ],
        out_specs=pl.BlockSpec((ZROWS_TC, WT, K, 128), lambda i: (i, 0, 0, 0)),
        out_shape=jax.ShapeDtypeStruct((B * H, WT, K, 128), jnp.float32),
    )(idx_k, zb_k)


def _renderer(idx_k, tab_t):
    mesh = plsc.VectorSubcoreMesh(
        core_axis_name="c", subcore_axis_name="s",
        num_cores=NC, num_subcores=NS)

    @functools.partial(
        pl.kernel,
        out_type=jax.ShapeDtypeStruct((B * (C + 1), PLANE), jnp.float32),
        mesh=mesh,
        compiler_params=pltpu.CompilerParams(
            needs_layout_passes=False, use_tc_tiling_on_sc=False),
        scratch_types=[
            pltpu.VMEM((P + L,), jnp.float32),
            pltpu.VMEM((8, 128), jnp.int32),
            pltpu.VMEM((1024,), jnp.int32),
            pltpu.VMEM((2, STEP), jnp.int32),
            pltpu.VMEM((2, STEP), jnp.float32),
            pltpu.VMEM_SHARED((PLANE,), jnp.int32),
            pltpu.SemaphoreType.DMA,
            pltpu.SemaphoreType.DMA,
            pltpu.SemaphoreType.DMA,
            pltpu.SemaphoreType.DMA,
            pltpu.SemaphoreType.DMA,
            pltpu.SemaphoreType.DMA,
        ],
    )
    def body(idx_hbm, tab_hbm, feat_hbm,
             plane_v, idx0_v, sel_v, selb_v, ob_v, sel_sh,
             psem, isem, ss0, ss1, os0, os1):
        core = lax.axis_index("c")
        tile = lax.axis_index("s")
        ssems = (ss0, ss1)
        osems = (os0, os1)
        cpt = TPI // NS

        first_cp = pltpu.async_copy(
            tab_hbm.at[tile, :], plane_v.at[pl.ds(0, P)], psem)

        def idx_read(b, ci):
            bh0 = b * H + (ci // WT) * 8
            return pltpu.make_async_copy(
                idx_hbm.at[pl.ds(bh0, 8), ci % WT, 0], idx0_v, isem)

        def phase1(b):
            c0 = tile * cpt
            arow = b * (C + 1) + C
            idx_read(b, c0).start()

            def chunk(ci, carry):
                idx_read(b, ci).wait()
                pl.when(ci > c0)(
                    lambda: pltpu.make_async_copy(
                        ob_v.at[0, pl.ds(0, 1024)],
                        feat_hbm.at[arow, pl.ds(0, 1024)],
                        osems[0]).wait())

                def grp(g8, c2):
                    for u in range(8):
                        g = g8 * 8 + u
                        hs = jnp.right_shift(g, 3)
                        wg = jnp.bitwise_and(g, 7) * L
                        idx0 = idx0_v[hs, pl.ds(wg, L)]
                        m = idx0 >= 0
                        sel_v[pl.ds(g * L, L)] = jnp.where(
                            m, jnp.minimum(idx0, P - 1), P)
                        ob_v[0, pl.ds(g * L, L)] = m.astype(jnp.float32)
                    return c2

                lax.fori_loop(0, 8, grp, 0)
                pltpu.sync_copy(sel_v, sel_sh.at[pl.ds(ci * 1024, 1024)])
                pltpu.async_copy(
                    ob_v.at[0, pl.ds(0, 1024)],
                    feat_hbm.at[arow, pl.ds(ci * 1024, 1024)], osems[0])
                pl.when(ci + 1 < c0 + cpt)(
                    lambda: idx_read(b, ci + 1).start())
                return carry

            lax.fori_loop(c0, c0 + cpt, chunk, 0)
            pltpu.make_async_copy(
                ob_v.at[0, pl.ds(0, 1024)],
                feat_hbm.at[arow, pl.ds(0, 1024)], osems[0]).wait()

        def start_sel(si, j):
            pltpu.async_copy(
                sel_sh.at[pl.ds(si * STEP, STEP)], selb_v.at[j], ssems[j])

        def gather_steps(row):
            start_sel(0, 0)
            start_sel(1, 1)

            def pair(sp, carry):
                for j in range(2):
                    si = sp * 2 + j
                    pltpu.make_async_copy(
                        sel_sh.at[pl.ds(si * STEP, STEP)], selb_v.at[j],
                        ssems[j]).wait()

                    @pl.when(sp > 0)
                    def _():
                        pltpu.make_async_copy(
                            ob_v.at[j],
                            feat_hbm.at[row, pl.ds(0, STEP)],
                            osems[j]).wait()

                    def grp(g8, c2):
                        for u in range(8):
                            g = g8 * 8 + u
                            s = selb_v[j, pl.ds(g * L, L)]
                            ob_v[j, pl.ds(g * L, L)] = (
                                plsc.load_gather(plane_v, [s]))
                        return c2

                    lax.fori_loop(0, STEP // L // 8, grp, 0)

                    @pl.when(si + 2 < NSTEPS)
                    def _():
                        start_sel(si + 2, j)

                    pltpu.async_copy(
                        ob_v.at[j],
                        feat_hbm.at[row, pl.ds(si * STEP, STEP)],
                        osems[j])
                return carry

            lax.fori_loop(0, NSTEPS // 2, pair, 0)
            for j in range(2):
                pltpu.make_async_copy(
                    ob_v.at[j], feat_hbm.at[row, pl.ds(0, STEP)],
                    osems[j]).wait()

        def phase2(b, chan, cp):
            if cp is not None:
                cp.wait()
                plane_v[pl.ds(P, L)] = jnp.zeros((L,), jnp.float32)
            gather_steps(b * (C + 1) + chan)

        b0 = NC * core
        b1 = b0 + 1

        phase1(b0)
        plsc.subcore_barrier()
        phase2(b0, tile, first_cp)
        cp2 = pltpu.async_copy(
            tab_hbm.at[NS + tile, :], plane_v.at[pl.ds(0, P)], psem)
        phase2(b0, NS + tile, cp2)
        plsc.subcore_barrier()
        phase1(b1)
        plsc.subcore_barrier()
        phase2(b1, NS + tile, None)
        cp3 = pltpu.async_copy(
            tab_hbm.at[tile, :], plane_v.at[pl.ds(0, P)], psem)
        phase2(b1, tile, cp3)

    return body(idx_k, tab_t)


def kernel(idx, features_packed, zbuf):
    idx_k = idx.reshape(B * H, WT, 128, K).transpose(0, 1, 3, 2)
    zb_k = zbuf.reshape(B * H, WT, 128, K).transpose(0, 1, 3, 2)
    tab_t = features_packed.T
    dep_k = _depth_tc(idx_k, zb_k)
    feat_p = _renderer(idx_k, tab_t)
    feat = (feat_p.reshape(B, C + 1, HT, WT, 8, 128)
            .transpose(0, 2, 4, 3, 5, 1)
            .reshape(B, H, W, C + 1))
    dep = (dep_k.reshape(B * H, WT, K, 128)
           .transpose(0, 1, 3, 2)
           .reshape(B, H, W, K))
    return feat, dep
